# baseline (speedup 1.0000x reference)
import jax
import jax.numpy as jnp
from jax import lax
from jax.experimental import pallas as pl
from jax.experimental.pallas import tpu as pltpu
from jax.experimental.pallas import tpu_sc as plsc

TOKENS = 16384
INPUT_DIM = 2048
NUM_EXPERTS = 16
K = 2
TILE = 512

NC = 2
NS = 16
L = 16
NW = NC * NS
CHUNK = TOKENS // NW
GROUPS = CHUNK // L


def _softmax_body(x_ref, w_ref, b_ref, probs_ref):
    logits = jnp.dot(x_ref[...], w_ref[...],
                     preferred_element_type=jnp.float32) + b_ref[...]
    m = jnp.max(logits, axis=1, keepdims=True)
    e = jnp.exp(logits - m)
    probs_ref[...] = e / jnp.sum(e, axis=1, keepdims=True)


def _route_body(probs_hbm, wts_hbm, idx_hbm, mask_hbm,
                probs_v, wts_v, idx_v, mask_v):
    wid = lax.axis_index("s") * NC + lax.axis_index("c")
    base = wid * CHUNK
    pltpu.sync_copy(probs_hbm.at[pl.ds(base, CHUNK)], probs_v)

    lanes = lax.iota(jnp.int32, L)
    zeros_i = jnp.zeros((L,), jnp.int32)
    ones_i = jnp.full((L,), 1, jnp.int32)

    def group(g, carry):
        rows = g * L + lanes
        rowsE = rows * NUM_EXPERTS
        rowsK = rows * K
        evecs = [jnp.full((L,), e, jnp.int32) for e in range(NUM_EXPERTS)]
        ps = [plsc.load_gather(probs_v, [rows, evecs[e]])
              for e in range(NUM_EXPERTS)]
        m1 = ps[0]
        i1 = zeros_i
        m2 = jnp.full((L,), -1.0, jnp.float32)
        i2 = zeros_i
        for e in range(1, NUM_EXPERTS):
            pe = ps[e]
            gt1 = pe > m1
            gt2 = pe > m2
            i2 = jnp.where(gt1, i1, jnp.where(gt2, evecs[e], i2))
            m2 = jnp.where(gt1, m1, jnp.where(gt2, pe, m2))
            i1 = jnp.where(gt1, evecs[e], i1)
            m1 = jnp.where(gt1, pe, m1)
        s = m1 + m2
        plsc.store_scatter(wts_v, [rows, zeros_i], m1 / s)
        plsc.store_scatter(wts_v, [rows, ones_i], m2 / s)
        plsc.store_scatter(idx_v, [rows, zeros_i], i1)
        plsc.store_scatter(idx_v, [rows, ones_i], i2)
        for e in range(NUM_EXPERTS):
            me = ((i1 == evecs[e]) | (i2 == evecs[e])).astype(jnp.float32)
            plsc.store_scatter(mask_v, [rows, evecs[e]], me)
        return carry

    lax.fori_loop(0, GROUPS, group, 0)
    pltpu.sync_copy(wts_v, wts_hbm.at[pl.ds(base, CHUNK)])
    pltpu.sync_copy(idx_v, idx_hbm.at[pl.ds(base, CHUNK)])
    pltpu.sync_copy(mask_v, mask_hbm.at[pl.ds(base, CHUNK)])


@jax.jit
def kernel(x, W, b):
    n_tiles = TOKENS // TILE
    probs = pl.pallas_call(
        _softmax_body,
        grid=(n_tiles,),
        in_specs=[
            pl.BlockSpec((TILE, INPUT_DIM), lambda i: (i, 0)),
            pl.BlockSpec((INPUT_DIM, NUM_EXPERTS), lambda i: (0, 0)),
            pl.BlockSpec((1, NUM_EXPERTS), lambda i: (0, 0)),
        ],
        out_specs=pl.BlockSpec((TILE, NUM_EXPERTS), lambda i: (i, 0)),
        out_shape=jax.ShapeDtypeStruct((TOKENS, NUM_EXPERTS), jnp.float32),
    )(x, W, b.reshape(1, NUM_EXPERTS))

    route = pl.kernel(
        _route_body,
        out_type=(
            jax.ShapeDtypeStruct((TOKENS, K), jnp.float32),
            jax.ShapeDtypeStruct((TOKENS, K), jnp.int32),
            jax.ShapeDtypeStruct((TOKENS, NUM_EXPERTS), jnp.float32),
        ),
        mesh=plsc.VectorSubcoreMesh(core_axis_name="c", subcore_axis_name="s"),
        compiler_params=pltpu.CompilerParams(needs_layout_passes=False,
                                             use_tc_tiling_on_sc=False),
        scratch_types=[
            pltpu.VMEM((CHUNK, NUM_EXPERTS), jnp.float32),
            pltpu.VMEM((CHUNK, K), jnp.float32),
            pltpu.VMEM((CHUNK, K), jnp.int32),
            pltpu.VMEM((CHUNK, NUM_EXPERTS), jnp.float32),
        ],
    )
    wts, idx, mask = route(probs)
    return (wts, idx, mask, probs)


# speedup vs baseline: 1.0609x; 1.0609x over previous
import jax
import jax.numpy as jnp
from jax import lax
from jax.experimental import pallas as pl
from jax.experimental.pallas import tpu as pltpu
from jax.experimental.pallas import tpu_sc as plsc

TOKENS = 16384
INPUT_DIM = 2048
NUM_EXPERTS = 16
K = 2
TILE = 512

NC = 2
NS = 16
L = 16
NW = NC * NS
CHUNK = TOKENS // NW
GROUPS = CHUNK // L


def _softmax_body(x_ref, w_ref, b_ref, probs_ref):
    logits = jnp.dot(x_ref[...], w_ref[...],
                     preferred_element_type=jnp.float32) + b_ref[...]
    m = jnp.max(logits, axis=1, keepdims=True)
    e = jnp.exp(logits - m)
    probs_ref[...] = e / jnp.sum(e, axis=1, keepdims=True)


SLAB = 128
NSLAB = CHUNK // SLAB
SGROUPS = SLAB // L


def _route_body(probs_hbm, wts_hbm, idx_hbm, mask_hbm,
                probs_v, wts_v, idx_v, mask_v):
    wid = lax.axis_index("s") * NC + lax.axis_index("c")
    base = wid * CHUNK

    lanes = lax.iota(jnp.int32, L)
    zeros_i = jnp.zeros((L,), jnp.int32)
    ones_i = jnp.full((L,), 1, jnp.int32)

    def slab(sl, carry):
        sbase = base + sl * SLAB
        pltpu.sync_copy(probs_hbm.at[pl.ds(sbase, SLAB)], probs_v)
        for g in range(SGROUPS):
            rows = g * L + lanes
            evecs = [jnp.full((L,), e, jnp.int32) for e in range(NUM_EXPERTS)]
            ps = [plsc.load_gather(probs_v, [rows, evecs[e]])
                  for e in range(NUM_EXPERTS)]
            m1 = ps[0]
            i1 = zeros_i
            m2 = jnp.full((L,), -1.0, jnp.float32)
            i2 = zeros_i
            for e in range(1, NUM_EXPERTS):
                pe = ps[e]
                gt1 = pe > m1
                gt2 = pe > m2
                i2 = jnp.where(gt1, i1, jnp.where(gt2, evecs[e], i2))
                m2 = jnp.where(gt1, m1, jnp.where(gt2, pe, m2))
                i1 = jnp.where(gt1, evecs[e], i1)
                m1 = jnp.where(gt1, pe, m1)
            s = m1 + m2
            plsc.store_scatter(wts_v, [rows, zeros_i], m1 / s)
            plsc.store_scatter(wts_v, [rows, ones_i], m2 / s)
            plsc.store_scatter(idx_v, [rows, zeros_i], i1)
            plsc.store_scatter(idx_v, [rows, ones_i], i2)
            for e in range(NUM_EXPERTS):
                me = ((i1 == evecs[e]) | (i2 == evecs[e])).astype(jnp.float32)
                plsc.store_scatter(mask_v, [rows, evecs[e]], me)
        pltpu.sync_copy(wts_v, wts_hbm.at[pl.ds(sbase, SLAB)])
        pltpu.sync_copy(idx_v, idx_hbm.at[pl.ds(sbase, SLAB)])
        pltpu.sync_copy(mask_v, mask_hbm.at[pl.ds(sbase, SLAB)])
        return carry

    lax.fori_loop(0, NSLAB, slab, 0)


@jax.jit
def kernel(x, W, b):
    n_tiles = TOKENS // TILE
    probs = pl.pallas_call(
        _softmax_body,
        grid=(n_tiles,),
        in_specs=[
            pl.BlockSpec((TILE, INPUT_DIM), lambda i: (i, 0)),
            pl.BlockSpec((INPUT_DIM, NUM_EXPERTS), lambda i: (0, 0)),
            pl.BlockSpec((1, NUM_EXPERTS), lambda i: (0, 0)),
        ],
        out_specs=pl.BlockSpec((TILE, NUM_EXPERTS), lambda i: (i, 0)),
        out_shape=jax.ShapeDtypeStruct((TOKENS, NUM_EXPERTS), jnp.float32),
    )(x, W, b.reshape(1, NUM_EXPERTS))

    route = pl.kernel(
        _route_body,
        out_type=(
            jax.ShapeDtypeStruct((TOKENS, K), jnp.float32),
            jax.ShapeDtypeStruct((TOKENS, K), jnp.int32),
            jax.ShapeDtypeStruct((TOKENS, NUM_EXPERTS), jnp.float32),
        ),
        mesh=plsc.VectorSubcoreMesh(core_axis_name="c", subcore_axis_name="s"),
        compiler_params=pltpu.CompilerParams(needs_layout_passes=False,
                                             use_tc_tiling_on_sc=True),
        scratch_types=[
            pltpu.VMEM((SLAB, NUM_EXPERTS), jnp.float32),
            pltpu.VMEM((SLAB, K), jnp.float32),
            pltpu.VMEM((SLAB, K), jnp.int32),
            pltpu.VMEM((SLAB, NUM_EXPERTS), jnp.float32),
        ],
    )
    wts, idx, mask = route(probs)
    return (wts, idx, mask, probs)


# speedup vs baseline: 1.6571x; 1.5620x over previous
import jax
import jax.numpy as jnp
from jax import lax
from jax.experimental import pallas as pl
from jax.experimental.pallas import tpu as pltpu
from jax.experimental.pallas import tpu_sc as plsc

TOKENS = 16384
INPUT_DIM = 2048
NUM_EXPERTS = 16
K = 2
TILE = 512

NC = 2
NS = 16
L = 16
NW = NC * NS
CHUNK = TOKENS // NW
GROUPS = CHUNK // L


def _softmax_t_body(x_ref, w_ref, b_ref, probs_ref):
    logits_t = lax.dot_general(
        w_ref[...], x_ref[...],
        dimension_numbers=(((0,), (1,)), ((), ())),
        preferred_element_type=jnp.float32) + b_ref[...]
    m = jnp.max(logits_t, axis=0, keepdims=True)
    e = jnp.exp(logits_t - m)
    probs_ref[...] = e / jnp.sum(e, axis=0, keepdims=True)


def _route_body(probs_hbm, wts_hbm, idx_hbm, mask_hbm,
                probs_v, wts_v, idx_v, mask_v):
    wid = lax.axis_index("s") * NC + lax.axis_index("c")
    base = wid * CHUNK
    pltpu.sync_copy(probs_hbm.at[:, pl.ds(base, CHUNK)], probs_v)

    def group(g, carry):
        off = g * L
        ps = [probs_v[e, pl.ds(off, L)] for e in range(NUM_EXPERTS)]
        evecs = [jnp.full((L,), e, jnp.int32) for e in range(NUM_EXPERTS)]
        m1 = ps[0]
        i1 = jnp.zeros((L,), jnp.int32)
        m2 = jnp.full((L,), -1.0, jnp.float32)
        i2 = jnp.zeros((L,), jnp.int32)
        for e in range(1, NUM_EXPERTS):
            pe = ps[e]
            gt1 = pe > m1
            gt2 = pe > m2
            i2 = jnp.where(gt1, i1, jnp.where(gt2, evecs[e], i2))
            m2 = jnp.where(gt1, m1, jnp.where(gt2, pe, m2))
            i1 = jnp.where(gt1, evecs[e], i1)
            m1 = jnp.where(gt1, pe, m1)
        s = m1 + m2
        wts_v[0, pl.ds(off, L)] = m1 / s
        wts_v[1, pl.ds(off, L)] = m2 / s
        idx_v[0, pl.ds(off, L)] = i1
        idx_v[1, pl.ds(off, L)] = i2
        for e in range(NUM_EXPERTS):
            me = ((i1 == evecs[e]) | (i2 == evecs[e])).astype(jnp.float32)
            mask_v[e, pl.ds(off, L)] = me
        return carry

    lax.fori_loop(0, GROUPS, group, 0)
    pltpu.sync_copy(wts_v, wts_hbm.at[:, pl.ds(base, CHUNK)])
    pltpu.sync_copy(idx_v, idx_hbm.at[:, pl.ds(base, CHUNK)])
    pltpu.sync_copy(mask_v, mask_hbm.at[:, pl.ds(base, CHUNK)])


@jax.jit
def kernel(x, W, b):
    n_tiles = TOKENS // TILE
    probs_t = pl.pallas_call(
        _softmax_t_body,
        grid=(n_tiles,),
        in_specs=[
            pl.BlockSpec((TILE, INPUT_DIM), lambda i: (i, 0)),
            pl.BlockSpec((INPUT_DIM, NUM_EXPERTS), lambda i: (0, 0)),
            pl.BlockSpec((NUM_EXPERTS, 1), lambda i: (0, 0)),
        ],
        out_specs=pl.BlockSpec((NUM_EXPERTS, TILE), lambda i: (0, i)),
        out_shape=jax.ShapeDtypeStruct((NUM_EXPERTS, TOKENS), jnp.float32),
    )(x, W, b.reshape(NUM_EXPERTS, 1))

    route = pl.kernel(
        _route_body,
        out_type=(
            jax.ShapeDtypeStruct((K, TOKENS), jnp.float32),
            jax.ShapeDtypeStruct((K, TOKENS), jnp.int32),
            jax.ShapeDtypeStruct((NUM_EXPERTS, TOKENS), jnp.float32),
        ),
        mesh=plsc.VectorSubcoreMesh(core_axis_name="c", subcore_axis_name="s"),
        compiler_params=pltpu.CompilerParams(needs_layout_passes=False,
                                             use_tc_tiling_on_sc=True),
        scratch_types=[
            pltpu.VMEM((NUM_EXPERTS, CHUNK), jnp.float32),
            pltpu.VMEM((K, CHUNK), jnp.float32),
            pltpu.VMEM((K, CHUNK), jnp.int32),
            pltpu.VMEM((NUM_EXPERTS, CHUNK), jnp.float32),
        ],
    )
    wts_t, idx_t, mask_t = route(probs_t)
    return (wts_t.T, idx_t.T, mask_t.T, probs_t.T)


# speedup vs baseline: 1.7216x; 1.0389x over previous
import jax
import jax.numpy as jnp
from jax import lax
from jax.experimental import pallas as pl
from jax.experimental.pallas import tpu as pltpu
from jax.experimental.pallas import tpu_sc as plsc

TOKENS = 16384
INPUT_DIM = 2048
NUM_EXPERTS = 16
K = 2
TILE = 512

NC = 2
NS = 16
L = 16
NW = NC * NS
CHUNK = TOKENS // NW
GROUPS = CHUNK // L


def _softmax_t_body(x_ref, w_ref, b_ref, probs_ref):
    logits_t = lax.dot_general(
        w_ref[...], x_ref[...],
        dimension_numbers=(((1,), (1,)), ((), ())),
        preferred_element_type=jnp.float32) + b_ref[...]
    m = jnp.max(logits_t, axis=0, keepdims=True)
    e = jnp.exp(logits_t - m)
    probs_ref[...] = e / jnp.sum(e, axis=0, keepdims=True)


def _route_body(probs_hbm, wts_hbm, idx_hbm, mask_hbm,
                probs_v, wts_v, idx_v, mask_v):
    wid = lax.axis_index("s") * NC + lax.axis_index("c")
    base = wid * CHUNK
    pltpu.sync_copy(probs_hbm.at[:, pl.ds(base, CHUNK)], probs_v)

    def group(g, carry):
        off = g * L
        ps = [probs_v[e, pl.ds(off, L)] for e in range(NUM_EXPERTS)]
        evecs = [jnp.full((L,), e, jnp.int32) for e in range(NUM_EXPERTS)]
        m1 = ps[0]
        i1 = jnp.zeros((L,), jnp.int32)
        m2 = jnp.full((L,), -1.0, jnp.float32)
        i2 = jnp.zeros((L,), jnp.int32)
        for e in range(1, NUM_EXPERTS):
            pe = ps[e]
            gt1 = pe > m1
            gt2 = pe > m2
            i2 = jnp.where(gt1, i1, jnp.where(gt2, evecs[e], i2))
            m2 = jnp.where(gt1, m1, jnp.where(gt2, pe, m2))
            i1 = jnp.where(gt1, evecs[e], i1)
            m1 = jnp.where(gt1, pe, m1)
        s = m1 + m2
        wts_v[0, pl.ds(off, L)] = m1 / s
        wts_v[1, pl.ds(off, L)] = m2 / s
        idx_v[0, pl.ds(off, L)] = i1
        idx_v[1, pl.ds(off, L)] = i2
        for e in range(NUM_EXPERTS):
            me = ((i1 == evecs[e]) | (i2 == evecs[e])).astype(jnp.float32)
            mask_v[e, pl.ds(off, L)] = me
        return carry

    lax.fori_loop(0, GROUPS, group, 0)
    pltpu.sync_copy(wts_v, wts_hbm.at[:, pl.ds(base, CHUNK)])
    pltpu.sync_copy(idx_v, idx_hbm.at[:, pl.ds(base, CHUNK)])
    pltpu.sync_copy(mask_v, mask_hbm.at[:, pl.ds(base, CHUNK)])


@jax.jit
def kernel(x, W, b):
    n_tiles = TOKENS // TILE
    probs_t = pl.pallas_call(
        _softmax_t_body,
        grid=(n_tiles,),
        in_specs=[
            pl.BlockSpec((TILE, INPUT_DIM), lambda i: (i, 0)),
            pl.BlockSpec((NUM_EXPERTS, INPUT_DIM), lambda i: (0, 0)),
            pl.BlockSpec((NUM_EXPERTS, 1), lambda i: (0, 0)),
        ],
        out_specs=pl.BlockSpec((NUM_EXPERTS, TILE), lambda i: (0, i)),
        out_shape=jax.ShapeDtypeStruct((NUM_EXPERTS, TOKENS), jnp.float32),
    )(x, W.T, b.reshape(NUM_EXPERTS, 1))

    route = pl.kernel(
        _route_body,
        out_type=(
            jax.ShapeDtypeStruct((K, TOKENS), jnp.float32),
            jax.ShapeDtypeStruct((K, TOKENS), jnp.int32),
            jax.ShapeDtypeStruct((NUM_EXPERTS, TOKENS), jnp.float32),
        ),
        mesh=plsc.VectorSubcoreMesh(core_axis_name="c", subcore_axis_name="s"),
        compiler_params=pltpu.CompilerParams(needs_layout_passes=False,
                                             use_tc_tiling_on_sc=True),
        scratch_types=[
            pltpu.VMEM((NUM_EXPERTS, CHUNK), jnp.float32),
            pltpu.VMEM((K, CHUNK), jnp.float32),
            pltpu.VMEM((K, CHUNK), jnp.int32),
            pltpu.VMEM((NUM_EXPERTS, CHUNK), jnp.float32),
        ],
    )
    wts_t, idx_t, mask_t = route(probs_t)
    return (wts_t.T, idx_t.T, mask_t.T, probs_t.T)


# speedup vs baseline: 1.9393x; 1.1264x over previous
import jax
import jax.numpy as jnp
from jax import lax
from jax.experimental import pallas as pl
from jax.experimental.pallas import tpu as pltpu
from jax.experimental.pallas import tpu_sc as plsc

TOKENS = 16384
INPUT_DIM = 2048
NUM_EXPERTS = 16
K = 2
TILE = 1024

NC = 2
NS = 16
L = 16
NW = NC * NS
CHUNK = TOKENS // NW
GROUPS = CHUNK // L


def _softmax_t_body(x_ref, w_ref, b_ref, probs_ref):
    logits_t = lax.dot_general(
        w_ref[...], x_ref[...],
        dimension_numbers=(((1,), (1,)), ((), ())),
        preferred_element_type=jnp.float32) + b_ref[...]
    m = jnp.max(logits_t, axis=0, keepdims=True)
    e = jnp.exp(logits_t - m)
    probs_ref[...] = e / jnp.sum(e, axis=0, keepdims=True)


def _route_body(probs_hbm, wts_hbm, idx_hbm, mask_hbm,
                probs_v, wts_v, idx_v, mask_v):
    wid = lax.axis_index("s") * NC + lax.axis_index("c")
    base = wid * CHUNK
    pltpu.sync_copy(probs_hbm.at[:, pl.ds(base, CHUNK)], probs_v)

    def group(g, carry):
        off = g * L
        ps = [probs_v[e, pl.ds(off, L)] for e in range(NUM_EXPERTS)]
        evecs = [jnp.full((L,), e, jnp.int32) for e in range(NUM_EXPERTS)]
        m1 = ps[0]
        i1 = jnp.zeros((L,), jnp.int32)
        m2 = jnp.full((L,), -1.0, jnp.float32)
        i2 = jnp.zeros((L,), jnp.int32)
        for e in range(1, NUM_EXPERTS):
            pe = ps[e]
            gt1 = pe > m1
            gt2 = pe > m2
            i2 = jnp.where(gt1, i1, jnp.where(gt2, evecs[e], i2))
            m2 = jnp.where(gt1, m1, jnp.where(gt2, pe, m2))
            i1 = jnp.where(gt1, evecs[e], i1)
            m1 = jnp.where(gt1, pe, m1)
        s = m1 + m2
        wts_v[0, pl.ds(off, L)] = m1 / s
        wts_v[1, pl.ds(off, L)] = m2 / s
        idx_v[0, pl.ds(off, L)] = i1
        idx_v[1, pl.ds(off, L)] = i2
        for e in range(NUM_EXPERTS):
            me = ((i1 == evecs[e]) | (i2 == evecs[e])).astype(jnp.float32)
            mask_v[e, pl.ds(off, L)] = me
        return carry

    lax.fori_loop(0, GROUPS, group, 0)
    pltpu.sync_copy(wts_v, wts_hbm.at[:, pl.ds(base, CHUNK)])
    pltpu.sync_copy(idx_v, idx_hbm.at[:, pl.ds(base, CHUNK)])
    pltpu.sync_copy(mask_v, mask_hbm.at[:, pl.ds(base, CHUNK)])


@jax.jit
def kernel(x, W, b):
    n_tiles = TOKENS // TILE
    probs_t = pl.pallas_call(
        _softmax_t_body,
        grid=(n_tiles,),
        in_specs=[
            pl.BlockSpec((TILE, INPUT_DIM), lambda i: (i, 0)),
            pl.BlockSpec((NUM_EXPERTS, INPUT_DIM), lambda i: (0, 0)),
            pl.BlockSpec((NUM_EXPERTS, 1), lambda i: (0, 0)),
        ],
        out_specs=pl.BlockSpec((NUM_EXPERTS, TILE), lambda i: (0, i)),
        out_shape=jax.ShapeDtypeStruct((NUM_EXPERTS, TOKENS), jnp.float32),
    )(x, W.T, b.reshape(NUM_EXPERTS, 1))

    route = pl.kernel(
        _route_body,
        out_type=(
            jax.ShapeDtypeStruct((K, TOKENS), jnp.float32),
            jax.ShapeDtypeStruct((K, TOKENS), jnp.int32),
            jax.ShapeDtypeStruct((NUM_EXPERTS, TOKENS), jnp.float32),
        ),
        mesh=plsc.VectorSubcoreMesh(core_axis_name="c", subcore_axis_name="s"),
        compiler_params=pltpu.CompilerParams(needs_layout_passes=False,
                                             use_tc_tiling_on_sc=True),
        scratch_types=[
            pltpu.VMEM((NUM_EXPERTS, CHUNK), jnp.float32),
            pltpu.VMEM((K, CHUNK), jnp.float32),
            pltpu.VMEM((K, CHUNK), jnp.int32),
            pltpu.VMEM((NUM_EXPERTS, CHUNK), jnp.float32),
        ],
    )
    wts_t, idx_t, mask_t = route(probs_t)
    return (wts_t.T, idx_t.T, mask_t.T, probs_t.T)
